# Initial kernel scaffold; baseline (speedup 1.0000x reference)
import jax
import jax.numpy as jnp
from jax.experimental import pallas as pl


def kernel(indices, table_0, table_1):
    raise NotImplementedError("write your pallas kernel here")


# speedup vs baseline: 1.5289x; 1.5289x over previous
import functools

import jax
from jax._src.config import enable_x64 as _x64_ctx
import jax.numpy as jnp
from jax import lax
from jax.experimental import pallas as pl
from jax.experimental.pallas import tpu as pltpu
from jax.experimental.pallas import tpu_sc as plsc

_ZCH = 1000000
_D = 32
_NC = 2
_NS = 16
_LANES = 16
_B = 16384
_H = 20
_PER_W = (_B * _H) // _NS
_ROWS = 128
_NJ = _PER_W // _ROWS
_C_HI = 435
_C_LO = 761


def _vconst(x):
    return jnp.full((_LANES,), x, dtype=jnp.int32)


def _hash16(v):
    t = lax.rem(v * _vconst(_C_HI), _vconst(_ZCH))
    return lax.rem(t * _vconst(1000) + v * _vconst(_C_LO), _vconst(_ZCH))


def _sc_body(ids_hbm, t0_hbm, t1_hbm, emb_hbm, rmp_hbm,
             ids_v, idx_v, bufs, gsem0, gsem1, ssem0, ssem1):
    f = lax.axis_index("c")
    w = lax.axis_index("s")

    pltpu.sync_copy(ids_hbm.at[f, w], ids_v)

    @pl.loop(0, _NJ)
    def _hash_row(j):
        for k in range(_ROWS // _LANES):
            sl = pl.ds(k * _LANES, _LANES)
            idx_v[j, sl] = _hash16(ids_v[j, sl])

    pltpu.sync_copy(idx_v, rmp_hbm.at[f, w])

    def gather_loop(tbl):
        gsems = (gsem0, gsem1)
        ssems = (ssem0, ssem1)

        def gstart(j, b):
            pltpu.async_copy(tbl.at[idx_v.at[j]], bufs.at[b], gsems[b])

        def gwait(b):
            pltpu.make_async_copy(tbl.at[idx_v.at[0]], bufs.at[b],
                                  gsems[b]).wait()

        def sstart(j, b):
            pltpu.async_copy(bufs.at[b], emb_hbm.at[f, w, j], ssems[b])

        def swait(b):
            pltpu.make_async_copy(bufs.at[b], emb_hbm.at[f, w, 0],
                                  ssems[b]).wait()

        gstart(0, 0)

        @pl.loop(0, _NJ, step=2)
        def _step(j):
            for b in range(2):
                jj = j + b
                nb = 1 - b

                @pl.when(jj + 1 < _NJ)
                def _():
                    @pl.when(jj >= 1)
                    def _():
                        swait(nb)

                    gstart(jj + 1, nb)

                gwait(b)
                sstart(jj, b)
        swait(0)
        swait(1)

    @pl.when(f == 0)
    def _():
        gather_loop(t0_hbm)

    @pl.when(f == 1)
    def _():
        gather_loop(t1_hbm)


_sc_lookup = functools.partial(
    pl.kernel,
    out_type=(
        jax.ShapeDtypeStruct((2, _NS, _NJ, _ROWS, _D), jnp.float32),
        jax.ShapeDtypeStruct((2, _NS, _NJ, _ROWS), jnp.int32),
    ),
    mesh=plsc.VectorSubcoreMesh(
        core_axis_name="c", subcore_axis_name="s",
        num_cores=_NC, num_subcores=_NS),
    compiler_params=pltpu.CompilerParams(use_tc_tiling_on_sc=False),
    scratch_types=(
        pltpu.VMEM((_NJ, _ROWS), jnp.int32),
        pltpu.VMEM((_NJ, _ROWS), jnp.int32),
        pltpu.VMEM((2, _ROWS, _D), jnp.float32),
        pltpu.SemaphoreType.DMA,
        pltpu.SemaphoreType.DMA,
        pltpu.SemaphoreType.DMA,
        pltpu.SemaphoreType.DMA,
    ),
)(_sc_body)


def kernel(indices, table_0, table_1):
    with _x64_ctx(False):
        ids32 = indices.astype(jnp.int32).reshape(2, _NS, _NJ, _ROWS)
        emb, rmp = _sc_lookup(ids32, table_0, table_1)
    embeddings = emb.reshape(2, _B, _H, _D)
    remapped = rmp.astype(indices.dtype).reshape(2, _B, _H)
    return embeddings, remapped


# speedup vs baseline: 1.5515x; 1.0148x over previous
import functools

import jax
from jax._src.config import enable_x64 as _x64_ctx
import jax.numpy as jnp
from jax import lax
from jax.experimental import pallas as pl
from jax.experimental.pallas import tpu as pltpu
from jax.experimental.pallas import tpu_sc as plsc

_ZCH = 1000000
_D = 32
_DP = 32
_NC = 2
_NS = 16
_LANES = 16
_B = 16384
_H = 20
_PER_W = (_B * _H) // _NS
_ROWS = 128
_NJ = _PER_W // _ROWS
_NB = 4
_C_HI = 435
_C_LO = 761


def _vconst(x):
    return jnp.full((_LANES,), x, dtype=jnp.int32)


def _hash16(v):
    t = lax.rem(v * _vconst(_C_HI), _vconst(_ZCH))
    return lax.rem(t * _vconst(1000) + v * _vconst(_C_LO), _vconst(_ZCH))


def _sc_body(ids_hbm, t0_hbm, t1_hbm, emb_hbm, rmp_hbm,
             ids_v, idx_v, bufs, gsem0, gsem1, gsem2, gsem3,
             ssem0, ssem1, ssem2, ssem3):
    f = lax.axis_index("c")
    w = lax.axis_index("s")
    gsems = (gsem0, gsem1, gsem2, gsem3)
    ssems = (ssem0, ssem1, ssem2, ssem3)

    pltpu.sync_copy(ids_hbm.at[f, w], ids_v)

    @pl.loop(0, _NJ)
    def _hash_row(j):
        for k in range(_ROWS // _LANES):
            sl = pl.ds(k * _LANES, _LANES)
            idx_v[j, sl] = _hash16(ids_v[j, sl])

    pltpu.sync_copy(idx_v, rmp_hbm.at[f, w])

    def gather_loop(tbl):
        def gstart(j, b):
            pltpu.async_copy(tbl.at[idx_v.at[j]], bufs.at[b], gsems[b])

        def gwait(b):
            pltpu.make_async_copy(tbl.at[idx_v.at[0]], bufs.at[b],
                                  gsems[b]).wait()

        def sstart(j, b):
            pltpu.async_copy(bufs.at[b, :, :_D], emb_hbm.at[f, w, j],
                             ssems[b])

        def swait(b):
            pltpu.make_async_copy(bufs.at[b, :, :_D], emb_hbm.at[f, w, 0],
                                  ssems[b]).wait()

        for p in range(2):
            gstart(p, p)

        @pl.loop(0, _NJ, step=_NB)
        def _step(j):
            for b in range(_NB):
                jj = j + b
                nb = (b + 2) % _NB

                @pl.when(jj + 2 < _NJ)
                def _():
                    @pl.when(jj >= 2)
                    def _():
                        swait(nb)

                    gstart(jj + 2, nb)

                gwait(b)
                sstart(jj, b)
        for p in range(_NB):
            swait(p)

    @pl.when(f == 0)
    def _():
        gather_loop(t0_hbm)

    @pl.when(f == 1)
    def _():
        gather_loop(t1_hbm)


_sc_lookup = functools.partial(
    pl.kernel,
    out_type=(
        jax.ShapeDtypeStruct((2, _NS, _NJ, _ROWS, _D), jnp.float32),
        jax.ShapeDtypeStruct((2, _NS, _NJ, _ROWS), jnp.int32),
    ),
    mesh=plsc.VectorSubcoreMesh(
        core_axis_name="c", subcore_axis_name="s",
        num_cores=_NC, num_subcores=_NS),
    compiler_params=pltpu.CompilerParams(use_tc_tiling_on_sc=False),
    scratch_types=(
        pltpu.VMEM((_NJ, _ROWS), jnp.int32),
        pltpu.VMEM((_NJ, _ROWS), jnp.int32),
        pltpu.VMEM((_NB, _ROWS, _DP), jnp.float32),
        pltpu.SemaphoreType.DMA,
        pltpu.SemaphoreType.DMA,
        pltpu.SemaphoreType.DMA,
        pltpu.SemaphoreType.DMA,
        pltpu.SemaphoreType.DMA,
        pltpu.SemaphoreType.DMA,
        pltpu.SemaphoreType.DMA,
        pltpu.SemaphoreType.DMA,
    ),
)(_sc_body)


def kernel(indices, table_0, table_1):
    with _x64_ctx(False):
        ids32 = indices.astype(jnp.int32).reshape(2, _NS, _NJ, _ROWS)
        emb, rmp = _sc_lookup(ids32, table_0, table_1)
    embeddings = emb.reshape(2, _B, _H, _D)
    remapped = rmp.astype(indices.dtype).reshape(2, _B, _H)
    return embeddings, remapped
